# baseline (speedup 1.0000x reference)
import functools

import jax
import jax.numpy as jnp
from jax import lax
from jax.experimental import pallas as pl
from jax.experimental.pallas import tpu as pltpu
from jax.experimental.pallas import tpu_sc as plsc

_VOCAB = 1000000
_EMBED = 64
_B = 4096
_QLEN = 20
_QPAD = 24
_TLEN = 200
_IDXW = _QPAD + _TLEN
_NW = 32
_BPW = _B // _NW
_CHUNK = 56
_VPAD = 1000008


_TBLK = 4096


def _tr_body(wt_ref, w128_ref, wsum_ref):
  i = pl.program_id(0)
  blk = wt_ref[...]
  eye = (lax.broadcasted_iota(jnp.int32, (_EMBED, _EMBED), 0) ==
         lax.broadcasted_iota(jnp.int32, (_EMBED, _EMBED), 1)
         ).astype(jnp.float32)
  tr = jnp.transpose(blk)
  del eye
  w128_ref[:, 0:_EMBED] = tr
  cid = lax.broadcasted_iota(jnp.int32, blk.shape, 1) + i * _TBLK
  psum = jnp.sum(jnp.where(cid < _VOCAB + 1, blk, 0.0), axis=1,
                 keepdims=True)

  @pl.when(i == 0)
  def _():
    wsum_ref[...] = jnp.zeros_like(wsum_ref)

  wsum_ref[...] += psum


def _transpose_wsum(WT):
  grid = (_VPAD + _TBLK - 1) // _TBLK
  return pl.pallas_call(
      _tr_body,
      out_shape=[
          jax.ShapeDtypeStruct((_VPAD, 128), jnp.float32),
          jax.ShapeDtypeStruct((_EMBED, 1), jnp.float32),
      ],
      grid=(grid,),
      in_specs=[pl.BlockSpec((_EMBED, _TBLK), lambda i: (0, i))],
      out_specs=[
          pl.BlockSpec((_TBLK, 128), lambda i: (i, 0)),
          pl.BlockSpec((_EMBED, 1), lambda i: (0, 0)),
      ],
  )(WT)


def _sc_pool(idx_hbm, w_hbm, qout_hbm, tout_hbm, idx_v, rows0_v, rows1_v,
             rows2_v, qout_v, tout_v, sem0, sem1, sem2):
  wid = lax.axis_index("s") * 2 + lax.axis_index("c")
  base = pl.multiple_of(wid * (_BPW * _IDXW), 8)
  pltpu.sync_copy(idx_hbm.at[pl.ds(base, _BPW * _IDXW)], idx_v)
  obase = pl.multiple_of(wid * (_BPW * _EMBED), 8)

  bufs = (rows0_v, rows1_v, rows2_v)
  sems = (sem0, sem1, sem2)

  def issue(b, buf, sem):
    off = pl.multiple_of(b * _IDXW, 8)
    pltpu.async_copy(w_hbm.at[idx_v.at[pl.ds(off, _IDXW)]], buf, sem)

  def drain(buf, sem):
    pltpu.make_async_copy(w_hbm.at[pl.ds(0, _IDXW)], buf, sem).wait()

  def rowsum(buf, r0, r1, g):
    acc = buf[r0, pl.ds(g * 16, 16)]
    for r in range(r0 + 1, r1):
      acc = acc + buf[r, pl.ds(g * 16, 16)]
    return acc

  def pool(b, buf):
    oof = pl.multiple_of(b * _EMBED, 8)
    for g in range(4):
      qout_v[pl.ds(oof + g * 16, 16)] = rowsum(buf, 0, _QPAD, g)
    for g in range(4):
      tout_v[pl.ds(oof + g * 16, 16)] = rowsum(buf, _QPAD, _IDXW, g)

  for u in range(3):
    issue(u, bufs[u], sems[u])

  def ring(g, _):
    for u in range(3):
      b = 3 * g + u
      drain(bufs[u], sems[u])
      pool(b, bufs[u])

      @pl.when(b + 3 < _BPW - 2)
      def _():
        issue(b + 3, bufs[u], sems[u])

    return 0

  lax.fori_loop(0, (_BPW - 2) // 3, ring, 0)
  for e in range(2):
    b = _BPW - 2 + e
    issue(b, bufs[e], sems[e])
    drain(bufs[e], sems[e])
    pool(b, bufs[e])
  pltpu.sync_copy(qout_v, qout_hbm.at[pl.ds(obase, _BPW * _EMBED)])
  pltpu.sync_copy(tout_v, tout_hbm.at[pl.ds(obase, _BPW * _EMBED)])


_sc_pool_call = functools.partial(
    pl.kernel,
    out_type=[
        jax.ShapeDtypeStruct((_B * _EMBED,), jnp.float32),
        jax.ShapeDtypeStruct((_B * _EMBED,), jnp.float32),
    ],
    mesh=plsc.VectorSubcoreMesh(core_axis_name="c", subcore_axis_name="s"),
    compiler_params=pltpu.CompilerParams(use_tc_tiling_on_sc=False),
    scratch_types=[
        pltpu.VMEM((_BPW * _IDXW,), jnp.int32),
        pltpu.VMEM((_IDXW, _EMBED), jnp.float32),
        pltpu.VMEM((_IDXW, _EMBED), jnp.float32),
        pltpu.VMEM((_IDXW, _EMBED), jnp.float32),
        pltpu.VMEM((_BPW * _EMBED,), jnp.float32),
        pltpu.VMEM((_BPW * _EMBED,), jnp.float32),
        pltpu.SemaphoreType.DMA,
        pltpu.SemaphoreType.DMA,
        pltpu.SemaphoreType.DMA,
    ],
)(_sc_pool)


def _combine_body(qraw_ref, traw_ref, wsum_ref, w0_ref, qlen_ref, tlen_ref,
                  s_ref, sneg_ref):
  w0 = w0_ref[...]
  wsum = wsum_ref[...] - w0
  qlen = qlen_ref[...].astype(jnp.float32)
  tlen = tlen_ref[...].astype(jnp.float32)
  q = (qraw_ref[...] - 4.0 * w0) / qlen
  t = traw_ref[...]
  s_ref[...] = jnp.sum((t / tlen) * q, axis=1, keepdims=True)
  sneg_ref[...] = jnp.sum((wsum - t) * q, axis=1, keepdims=True) * (
      1.0 / float(_VOCAB))


def _combine(qraw, traw, wsum, w0, qlen, tlen):
  return pl.pallas_call(
      _combine_body,
      out_shape=[
          jax.ShapeDtypeStruct((_B, 1), jnp.float32),
          jax.ShapeDtypeStruct((_B, 1), jnp.float32),
      ],
  )(qraw, traw, wsum, w0, qlen, tlen)


def kernel(q, q_len, text, text_len, W):
  idx = 2 * jnp.concatenate(
      [q, jnp.zeros((_B, _QPAD - _QLEN), jnp.int32), text],
      axis=1).reshape(-1)
  w128, wsum = _transpose_wsum(W.T)
  qraw, traw = _sc_pool_call(idx, w128.reshape(2 * _VPAD, _EMBED))
  w0 = lax.slice(W, (0, 0), (1, _EMBED))
  s, s_neg = _combine(qraw.reshape(_B, _EMBED), traw.reshape(_B, _EMBED),
                      wsum.reshape(1, _EMBED), w0, q_len.reshape(_B, 1),
                      text_len.reshape(_B, 1))
  return (s.reshape(-1), s_neg.reshape(-1))


# speedup vs baseline: 1.0237x; 1.0237x over previous
import functools

import jax
import jax.numpy as jnp
from jax import lax
from jax.experimental import pallas as pl
from jax.experimental.pallas import tpu as pltpu
from jax.experimental.pallas import tpu_sc as plsc

_VOCAB = 1000000
_EMBED = 64
_B = 4096
_QLEN = 20
_QPAD = 24
_TLEN = 200
_IDXW = _QPAD + _TLEN
_NW = 32
_BPW = _B // _NW
_CHUNK = 56
_VPAD = 1000008


_TBLK = 4096


def _tr_body(wt_ref, w128_ref, wsum_ref):
  i = pl.program_id(0)
  blk = wt_ref[...]
  eye = (lax.broadcasted_iota(jnp.int32, (_EMBED, _EMBED), 0) ==
         lax.broadcasted_iota(jnp.int32, (_EMBED, _EMBED), 1)
         ).astype(jnp.float32)
  tr = jnp.transpose(blk)
  del eye
  w128_ref[:, 0:_EMBED] = tr
  cid = lax.broadcasted_iota(jnp.int32, blk.shape, 1) + i * _TBLK
  psum = jnp.sum(jnp.where(cid < _VOCAB + 1, blk, 0.0), axis=1,
                 keepdims=True)

  @pl.when(i == 0)
  def _():
    wsum_ref[...] = jnp.zeros_like(wsum_ref)

  wsum_ref[...] += psum


def _transpose_wsum(WT):
  grid = (_VPAD + _TBLK - 1) // _TBLK
  return pl.pallas_call(
      _tr_body,
      out_shape=[
          jax.ShapeDtypeStruct((_VPAD, 128), jnp.float32),
          jax.ShapeDtypeStruct((_EMBED, 1), jnp.float32),
      ],
      grid=(grid,),
      in_specs=[pl.BlockSpec((_EMBED, _TBLK), lambda i: (0, i))],
      out_specs=[
          pl.BlockSpec((_TBLK, 128), lambda i: (i, 0)),
          pl.BlockSpec((_EMBED, 1), lambda i: (0, 0)),
      ],
  )(WT)


def _sc_pool(idx_hbm, w_hbm, qout_hbm, tout_hbm, idx_v, rows0_v, rows1_v,
             qout_v, tout_v, sem0, sem1):
  wid = lax.axis_index("s") * 2 + lax.axis_index("c")
  base = pl.multiple_of(wid * (_BPW * _IDXW), 8)
  pltpu.sync_copy(idx_hbm.at[pl.ds(base, _BPW * _IDXW)], idx_v)
  obase = pl.multiple_of(wid * (_BPW * _EMBED), 8)

  bufs = (rows0_v, rows1_v)
  sems = (sem0, sem1)

  def issue(b, buf, sem):
    off = pl.multiple_of(b * _IDXW, 8)
    pltpu.async_copy(w_hbm.at[idx_v.at[pl.ds(off, _IDXW)]], buf, sem)

  def drain(buf, sem):
    pltpu.make_async_copy(w_hbm.at[pl.ds(0, _IDXW)], buf, sem).wait()

  def rowsum(buf, r0, r1, g):
    acc = buf[r0, pl.ds(g * 16, 16)]
    for r in range(r0 + 1, r1):
      acc = acc + buf[r, pl.ds(g * 16, 16)]
    return acc

  def pool(b, buf):
    oof = pl.multiple_of(b * _EMBED, 8)
    for g in range(4):
      qout_v[pl.ds(oof + g * 16, 16)] = rowsum(buf, 0, _QPAD, g)
    for g in range(4):
      tout_v[pl.ds(oof + g * 16, 16)] = rowsum(buf, _QPAD, _IDXW, g)

  issue(0, bufs[0], sems[0])
  issue(1, bufs[1], sems[1])

  def ring(g, _):
    for u in range(2):
      b = 2 * g + u
      drain(bufs[u], sems[u])
      pool(b, bufs[u])

      @pl.when(b + 2 < _BPW)
      def _():
        issue(b + 2, bufs[u], sems[u])

    return 0

  lax.fori_loop(0, _BPW // 2, ring, 0)
  pltpu.sync_copy(qout_v, qout_hbm.at[pl.ds(obase, _BPW * _EMBED)])
  pltpu.sync_copy(tout_v, tout_hbm.at[pl.ds(obase, _BPW * _EMBED)])


_sc_pool_call = functools.partial(
    pl.kernel,
    out_type=[
        jax.ShapeDtypeStruct((_B * _EMBED,), jnp.float32),
        jax.ShapeDtypeStruct((_B * _EMBED,), jnp.float32),
    ],
    mesh=plsc.VectorSubcoreMesh(core_axis_name="c", subcore_axis_name="s"),
    compiler_params=pltpu.CompilerParams(use_tc_tiling_on_sc=False),
    scratch_types=[
        pltpu.VMEM((_BPW * _IDXW,), jnp.int32),
        pltpu.VMEM((_IDXW, _EMBED), jnp.float32),
        pltpu.VMEM((_IDXW, _EMBED), jnp.float32),
        pltpu.VMEM((_BPW * _EMBED,), jnp.float32),
        pltpu.VMEM((_BPW * _EMBED,), jnp.float32),
        pltpu.SemaphoreType.DMA,
        pltpu.SemaphoreType.DMA,
    ],
)(_sc_pool)


def _combine_body(qraw_ref, traw_ref, wsum_ref, w0_ref, qlen_ref, tlen_ref,
                  s_ref, sneg_ref):
  w0 = w0_ref[...]
  wsum = wsum_ref[...] - w0
  qlen = qlen_ref[...].astype(jnp.float32)
  tlen = tlen_ref[...].astype(jnp.float32)
  q = (qraw_ref[...] - 4.0 * w0) / qlen
  t = traw_ref[...]
  s_ref[...] = jnp.sum((t / tlen) * q, axis=1, keepdims=True)
  sneg_ref[...] = jnp.sum((wsum - t) * q, axis=1, keepdims=True) * (
      1.0 / float(_VOCAB))


def _combine(qraw, traw, wsum, w0, qlen, tlen):
  return pl.pallas_call(
      _combine_body,
      out_shape=[
          jax.ShapeDtypeStruct((_B, 1), jnp.float32),
          jax.ShapeDtypeStruct((_B, 1), jnp.float32),
      ],
  )(qraw, traw, wsum, w0, qlen, tlen)


def kernel(q, q_len, text, text_len, W):
  idx = 2 * jnp.concatenate(
      [q, jnp.zeros((_B, _QPAD - _QLEN), jnp.int32), text],
      axis=1).reshape(-1)
  w128, wsum = _transpose_wsum(W.T)
  qraw, traw = _sc_pool_call(idx, w128.reshape(2 * _VPAD, _EMBED))
  w0 = lax.slice(W, (0, 0), (1, _EMBED))
  s, s_neg = _combine(qraw.reshape(_B, _EMBED), traw.reshape(_B, _EMBED),
                      wsum.reshape(1, _EMBED), w0, q_len.reshape(_B, 1),
                      text_len.reshape(_B, 1))
  return (s.reshape(-1), s_neg.reshape(-1))
